# Initial kernel scaffold; baseline (speedup 1.0000x reference)
import jax
import jax.numpy as jnp
from jax.experimental import pallas as pl


def kernel(x, edge_index, W1, b1, W2, b2):
    raise NotImplementedError("write your pallas kernel here")


# speedup vs baseline: 12.7554x; 12.7554x over previous
import functools

import jax
import jax.numpy as jnp
from jax import lax
from jax.experimental import pallas as pl
from jax.experimental.pallas import tpu as pltpu
from jax.experimental.pallas import tpu_sc as plsc

N = 10000
E = 320000
NP = 10240
EP = 327680
CH = 128
NW = 32
NCH = EP // (NW * CH)
NSUB = 16
RPT = NP // NSUB

_mesh = plsc.VectorSubcoreMesh(core_axis_name="c", subcore_axis_name="s")
_sc_params = pltpu.CompilerParams(use_tc_tiling_on_sc=False)


def _deg_body(dst_hbm, ones_hbm, zeros_hbm, out_hbm, idx_v, ones_v, acc):
    cid = lax.axis_index("c")
    sid = lax.axis_index("s")
    r0 = pl.multiple_of(sid * RPT, RPT)
    pltpu.sync_copy(zeros_hbm.at[pl.ds(r0, RPT)], acc.at[pl.ds(r0, RPT)])
    pltpu.sync_copy(ones_hbm, ones_v)
    plsc.subcore_barrier()
    w = cid * NSUB + sid

    def body(j, carry):
        base = pl.multiple_of((w * NCH + j) * CH, CH)
        pltpu.sync_copy(dst_hbm.at[pl.ds(base, CH)], idx_v)
        pltpu.sync_copy(ones_v, acc.at[idx_v], add=True)
        return carry

    lax.fori_loop(0, NCH, body, 0)
    plsc.subcore_barrier()
    pltpu.sync_copy(acc.at[pl.ds(r0, RPT)], out_hbm.at[cid, pl.ds(r0, RPT)])


_deg_call = pl.kernel(
    _deg_body,
    out_type=jax.ShapeDtypeStruct((2, NP, 16), jnp.float32),
    mesh=_mesh,
    scratch_types=[
        pltpu.VMEM((CH,), jnp.int32),
        pltpu.VMEM((CH, 16), jnp.float32),
        pltpu.VMEM_SHARED((NP, 16), jnp.float32),
    ],
    compiler_params=_sc_params,
)


def _make_agg(W):
    def body_fn(g_hbm, src_hbm, dst_hbm, zeros_hbm, out_hbm,
                sidx, didx, rows, acc, sem):
        cid = lax.axis_index("c")
        sid = lax.axis_index("s")
        r0 = pl.multiple_of(sid * RPT, RPT)
        pltpu.sync_copy(zeros_hbm.at[pl.ds(r0, RPT)], acc.at[pl.ds(r0, RPT)])
        plsc.subcore_barrier()
        w = cid * NSUB + sid

        def body(j, carry):
            base = pl.multiple_of((w * NCH + j) * CH, CH)
            pltpu.sync_copy(src_hbm.at[pl.ds(base, CH)], sidx)
            pltpu.sync_copy(dst_hbm.at[pl.ds(base, CH)], didx)
            pltpu.async_copy(g_hbm.at[sidx], rows, sem).wait()
            pltpu.sync_copy(rows, acc.at[didx], add=True)
            return carry

        lax.fori_loop(0, NCH, body, 0)
        plsc.subcore_barrier()
        pltpu.sync_copy(acc.at[pl.ds(r0, RPT)], out_hbm.at[cid, pl.ds(r0, RPT)])

    return pl.kernel(
        body_fn,
        out_type=jax.ShapeDtypeStruct((2, NP, W), jnp.float32),
        mesh=_mesh,
        scratch_types=[
            pltpu.VMEM((CH,), jnp.int32),
            pltpu.VMEM((CH,), jnp.int32),
            pltpu.VMEM((CH, W), jnp.float32),
            pltpu.VMEM_SHARED((NP, W), jnp.float32),
            pltpu.SemaphoreType.DMA,
        ],
        compiler_params=_sc_params,
    )


_agg64 = _make_agg(64)
_agg32 = _make_agg(32)


R = 1024
GRID = NP // R


def _dis(d_ref):
    deg = d_ref[0, :, 0:1] + d_ref[1, :, 0:1] + 1.0
    return lax.rsqrt(deg)


def _l1_body(x_ref, w_ref, d_ref, o_ref):
    h = jnp.dot(x_ref[...], w_ref[...], preferred_element_type=jnp.float32)
    o_ref[...] = h * _dis(d_ref)


def _l1(xp, W1, degp):
    return pl.pallas_call(
        _l1_body,
        grid=(GRID,),
        in_specs=[
            pl.BlockSpec((R, 128), lambda i: (i, 0)),
            pl.BlockSpec((128, 64), lambda i: (0, 0)),
            pl.BlockSpec((2, R, 16), lambda i: (0, i, 0)),
        ],
        out_specs=pl.BlockSpec((R, 64), lambda i: (i, 0)),
        out_shape=jax.ShapeDtypeStruct((NP, 64), jnp.float32),
    )(xp, W1, degp)


def _l2_body(a_ref, g_ref, d_ref, b_ref, w_ref, o_ref):
    dis = _dis(d_ref)
    agg = a_ref[0] + a_ref[1] + g_ref[...]
    h = jnp.maximum(agg * dis + b_ref[...], 0.0)
    o_ref[...] = jnp.dot(h, w_ref[...], preferred_element_type=jnp.float32) * dis


def _l2(a1, g1, degp, b1, W2):
    return pl.pallas_call(
        _l2_body,
        grid=(GRID,),
        in_specs=[
            pl.BlockSpec((2, R, 64), lambda i: (0, i, 0)),
            pl.BlockSpec((R, 64), lambda i: (i, 0)),
            pl.BlockSpec((2, R, 16), lambda i: (0, i, 0)),
            pl.BlockSpec((1, 64), lambda i: (0, 0)),
            pl.BlockSpec((64, 32), lambda i: (0, 0)),
        ],
        out_specs=pl.BlockSpec((R, 32), lambda i: (i, 0)),
        out_shape=jax.ShapeDtypeStruct((NP, 32), jnp.float32),
    )(a1, g1, degp, b1, W2)


def _l3_body(a_ref, g_ref, d_ref, b_ref, o_ref):
    dis = _dis(d_ref)
    agg = a_ref[0] + a_ref[1] + g_ref[...]
    o_ref[...] = jnp.maximum(agg * dis + b_ref[...], 0.0)


def _l3(a2, g2, degp, b2):
    return pl.pallas_call(
        _l3_body,
        grid=(GRID,),
        in_specs=[
            pl.BlockSpec((2, R, 32), lambda i: (0, i, 0)),
            pl.BlockSpec((R, 32), lambda i: (i, 0)),
            pl.BlockSpec((2, R, 16), lambda i: (0, i, 0)),
            pl.BlockSpec((1, 32), lambda i: (0, 0)),
        ],
        out_specs=pl.BlockSpec((R, 32), lambda i: (i, 0)),
        out_shape=jax.ShapeDtypeStruct((NP, 32), jnp.float32),
    )(a2, g2, degp, b2)


def kernel(x, edge_index, W1, b1, W2, b2):
    ei = edge_index.astype(jnp.int32)
    pad_idx = jnp.full((EP - E,), N, jnp.int32)
    src = jnp.concatenate([ei[0], pad_idx])
    dst = jnp.concatenate([ei[1], pad_idx])
    xp = jnp.zeros((NP, 128), jnp.float32).at[:N].set(x)
    ones16 = jnp.ones((CH, 16), jnp.float32)
    z16 = jnp.zeros((NP, 16), jnp.float32)
    z64 = jnp.zeros((NP, 64), jnp.float32)
    z32 = jnp.zeros((NP, 32), jnp.float32)

    degp = _deg_call(dst, ones16, z16)
    g1 = _l1(xp, W1, degp)
    a1 = _agg64(g1, src, dst, z64)
    g2 = _l2(a1, g1, degp, b1.reshape(1, 64), W2)
    a2 = _agg32(g2, src, dst, z32)
    zp = _l3(a2, g2, degp, b2.reshape(1, 32))
    return zp[:N]


# speedup vs baseline: 15.1630x; 1.1888x over previous
import functools

import jax
import jax.numpy as jnp
from jax import lax
from jax.experimental import pallas as pl
from jax.experimental.pallas import tpu as pltpu
from jax.experimental.pallas import tpu_sc as plsc

N = 10000
E = 320000
NP = 10240
EP = 327680
CH = 128
NW = 32
NCH = EP // (NW * CH)
NSUB = 16
RPT = NP // NSUB

_mesh = plsc.VectorSubcoreMesh(core_axis_name="c", subcore_axis_name="s")
_sc_params = pltpu.CompilerParams(use_tc_tiling_on_sc=False)


def _deg_body(dst_hbm, ones_hbm, zeros_hbm, out_hbm, idx_a, idx_b, ones_v,
              acc, isem_a, isem_b):
    cid = lax.axis_index("c")
    sid = lax.axis_index("s")
    r0 = pl.multiple_of(sid * RPT, RPT)
    pltpu.sync_copy(zeros_hbm.at[pl.ds(r0, RPT)], acc.at[pl.ds(r0, RPT)])
    pltpu.sync_copy(ones_hbm, ones_v)
    plsc.subcore_barrier()
    w = cid * NSUB + sid

    def chunk_src(j):
        jj = lax.min(j, NCH - 1)
        base = pl.multiple_of((w * NCH + jj) * CH, CH)
        return dst_hbm.at[pl.ds(base, CH)]

    pltpu.sync_copy(chunk_src(0), idx_a)

    def stage(c, idx_cur, idx_nxt, sem_nxt):
        pltpu.async_copy(chunk_src(c + 1), idx_nxt, sem_nxt)
        pltpu.sync_copy(ones_v, acc.at[idx_cur], add=True)
        pltpu.make_async_copy(chunk_src(c + 1), idx_nxt, sem_nxt).wait()

    def body(p, carry):
        c = p * 2
        stage(c, idx_a, idx_b, isem_b)
        stage(c + 1, idx_b, idx_a, isem_a)
        return carry

    lax.fori_loop(0, NCH // 2, body, 0)
    plsc.subcore_barrier()
    pltpu.sync_copy(acc.at[pl.ds(r0, RPT)], out_hbm.at[cid, pl.ds(r0, RPT)])


_deg_call = pl.kernel(
    _deg_body,
    out_type=jax.ShapeDtypeStruct((2, NP, 16), jnp.float32),
    mesh=_mesh,
    scratch_types=[
        pltpu.VMEM((CH,), jnp.int32),
        pltpu.VMEM((CH,), jnp.int32),
        pltpu.VMEM((CH, 16), jnp.float32),
        pltpu.VMEM_SHARED((NP, 16), jnp.float32),
        pltpu.SemaphoreType.DMA,
        pltpu.SemaphoreType.DMA,
    ],
    compiler_params=_sc_params,
)


def _make_agg(W):
    def body_fn(g_hbm, pk_hbm, zeros_hbm, out_hbm,
                idx_a, idx_b, rows_a, rows_b, acc,
                isem_a, isem_b, gsem_a, gsem_b):
        cid = lax.axis_index("c")
        sid = lax.axis_index("s")
        r0 = pl.multiple_of(sid * RPT, RPT)
        pltpu.sync_copy(zeros_hbm.at[pl.ds(r0, RPT)], acc.at[pl.ds(r0, RPT)])
        plsc.subcore_barrier()
        w = cid * NSUB + sid

        def chunk_src(j):
            jj = lax.min(j, NCH - 1)
            return pk_hbm.at[w * NCH + jj]

        pltpu.sync_copy(chunk_src(0), idx_a)
        pltpu.async_copy(g_hbm.at[idx_a.at[0]], rows_a, gsem_a)

        def stage(c, idx_cur, idx_nxt, rows_cur, rows_nxt, isem_nxt,
                  gsem_cur, gsem_nxt):
            pltpu.async_copy(chunk_src(c + 1), idx_nxt, isem_nxt)
            pltpu.make_async_copy(g_hbm.at[idx_cur.at[0]], rows_cur,
                                  gsem_cur).wait()
            pltpu.make_async_copy(chunk_src(c + 1), idx_nxt, isem_nxt).wait()
            pltpu.async_copy(g_hbm.at[idx_nxt.at[0]], rows_nxt, gsem_nxt)
            pltpu.sync_copy(rows_cur, acc.at[idx_cur.at[1]], add=True)

        def body(p, carry):
            c = p * 2
            stage(c, idx_a, idx_b, rows_a, rows_b, isem_b, gsem_a, gsem_b)
            stage(c + 1, idx_b, idx_a, rows_b, rows_a, isem_a, gsem_b, gsem_a)
            return carry

        lax.fori_loop(0, NCH // 2, body, 0)
        pltpu.make_async_copy(g_hbm.at[idx_a.at[0]], rows_a, gsem_a).wait()
        plsc.subcore_barrier()
        pltpu.sync_copy(acc.at[pl.ds(r0, RPT)], out_hbm.at[cid, pl.ds(r0, RPT)])

    return pl.kernel(
        body_fn,
        out_type=jax.ShapeDtypeStruct((2, NP, W), jnp.float32),
        mesh=_mesh,
        scratch_types=[
            pltpu.VMEM((2, CH), jnp.int32),
            pltpu.VMEM((2, CH), jnp.int32),
            pltpu.VMEM((CH, W), jnp.float32),
            pltpu.VMEM((CH, W), jnp.float32),
            pltpu.VMEM_SHARED((NP, W), jnp.float32),
            pltpu.SemaphoreType.DMA,
            pltpu.SemaphoreType.DMA,
            pltpu.SemaphoreType.DMA,
            pltpu.SemaphoreType.DMA,
        ],
        compiler_params=_sc_params,
    )


_agg64 = _make_agg(64)
_agg32 = _make_agg(32)


R = 1024
GRID = NP // R


def _dis(d_ref):
    deg = d_ref[0, :, 0:1] + d_ref[1, :, 0:1] + 1.0
    return lax.rsqrt(deg)


def _l1_body(x_ref, w_ref, d_ref, o_ref):
    h = jnp.dot(x_ref[...], w_ref[...], preferred_element_type=jnp.float32)
    o_ref[...] = h * _dis(d_ref)


def _l1(xp, W1, degp):
    return pl.pallas_call(
        _l1_body,
        grid=(GRID,),
        in_specs=[
            pl.BlockSpec((R, 128), lambda i: (i, 0)),
            pl.BlockSpec((128, 64), lambda i: (0, 0)),
            pl.BlockSpec((2, R, 16), lambda i: (0, i, 0)),
        ],
        out_specs=pl.BlockSpec((R, 64), lambda i: (i, 0)),
        out_shape=jax.ShapeDtypeStruct((NP, 64), jnp.float32),
    )(xp, W1, degp)


def _l2_body(a_ref, g_ref, d_ref, b_ref, w_ref, o_ref):
    dis = _dis(d_ref)
    agg = a_ref[0] + a_ref[1] + g_ref[...]
    h = jnp.maximum(agg * dis + b_ref[...], 0.0)
    o_ref[...] = jnp.dot(h, w_ref[...], preferred_element_type=jnp.float32) * dis


def _l2(a1, g1, degp, b1, W2):
    return pl.pallas_call(
        _l2_body,
        grid=(GRID,),
        in_specs=[
            pl.BlockSpec((2, R, 64), lambda i: (0, i, 0)),
            pl.BlockSpec((R, 64), lambda i: (i, 0)),
            pl.BlockSpec((2, R, 16), lambda i: (0, i, 0)),
            pl.BlockSpec((1, 64), lambda i: (0, 0)),
            pl.BlockSpec((64, 32), lambda i: (0, 0)),
        ],
        out_specs=pl.BlockSpec((R, 32), lambda i: (i, 0)),
        out_shape=jax.ShapeDtypeStruct((NP, 32), jnp.float32),
    )(a1, g1, degp, b1, W2)


def _l3_body(a_ref, g_ref, d_ref, b_ref, o_ref):
    dis = _dis(d_ref)
    agg = a_ref[0] + a_ref[1] + g_ref[...]
    o_ref[...] = jnp.maximum(agg * dis + b_ref[...], 0.0)


def _l3(a2, g2, degp, b2):
    return pl.pallas_call(
        _l3_body,
        grid=(GRID,),
        in_specs=[
            pl.BlockSpec((2, R, 32), lambda i: (0, i, 0)),
            pl.BlockSpec((R, 32), lambda i: (i, 0)),
            pl.BlockSpec((2, R, 16), lambda i: (0, i, 0)),
            pl.BlockSpec((1, 32), lambda i: (0, 0)),
        ],
        out_specs=pl.BlockSpec((R, 32), lambda i: (i, 0)),
        out_shape=jax.ShapeDtypeStruct((NP, 32), jnp.float32),
    )(a2, g2, degp, b2)


def kernel(x, edge_index, W1, b1, W2, b2):
    ei = edge_index.astype(jnp.int32)
    pad_idx = jnp.full((EP - E,), N, jnp.int32)
    src = jnp.concatenate([ei[0], pad_idx])
    dst = jnp.concatenate([ei[1], pad_idx])
    pk = jnp.stack([src, dst], 0).reshape(2, EP // CH, CH).transpose(1, 0, 2)
    xp = jnp.zeros((NP, 128), jnp.float32).at[:N].set(x)
    ones16 = jnp.ones((CH, 16), jnp.float32)
    z16 = jnp.zeros((NP, 16), jnp.float32)
    z64 = jnp.zeros((NP, 64), jnp.float32)
    z32 = jnp.zeros((NP, 32), jnp.float32)

    degp = _deg_call(dst, ones16, z16)
    g1 = _l1(xp, W1, degp)
    a1 = _agg64(g1, pk, z64)
    g2 = _l2(a1, g1, degp, b1.reshape(1, 64), W2)
    a2 = _agg32(g2, pk, z32)
    zp = _l3(a2, g2, degp, b2.reshape(1, 32))
    return zp[:N]


# speedup vs baseline: 28.6911x; 1.8922x over previous
import functools

import jax
import jax.numpy as jnp
from jax import lax
from jax.experimental import pallas as pl
from jax.experimental.pallas import tpu as pltpu
from jax.experimental.pallas import tpu_sc as plsc

N = 10000
E = 320000
NP = 10240
EP = 327680
CH = 128
NW = 32
NCH = EP // (NW * CH)
NSUB = 16
RPT = NP // NSUB

_mesh = plsc.VectorSubcoreMesh(core_axis_name="c", subcore_axis_name="s")
_sc_params = pltpu.CompilerParams(use_tc_tiling_on_sc=False)


def _deg_body(dst_hbm, ones_hbm, zeros_hbm, out_hbm, idx_a, idx_b, ones_v,
              acc, isem_a, isem_b):
    cid = lax.axis_index("c")
    sid = lax.axis_index("s")
    r0 = pl.multiple_of(sid * RPT, RPT)
    pltpu.sync_copy(zeros_hbm.at[pl.ds(r0, RPT)], acc.at[pl.ds(r0, RPT)])
    pltpu.sync_copy(ones_hbm, ones_v)
    plsc.subcore_barrier()
    w = cid * NSUB + sid

    def chunk_src(j):
        jj = lax.min(j, NCH - 1)
        base = pl.multiple_of((w * NCH + jj) * CH, CH)
        return dst_hbm.at[pl.ds(base, CH)]

    pltpu.sync_copy(chunk_src(0), idx_a)

    def stage(c, idx_cur, idx_nxt, sem_nxt):
        pltpu.async_copy(chunk_src(c + 1), idx_nxt, sem_nxt)
        pltpu.sync_copy(ones_v, acc.at[idx_cur], add=True)
        pltpu.make_async_copy(chunk_src(c + 1), idx_nxt, sem_nxt).wait()

    def body(p, carry):
        c = p * 2
        stage(c, idx_a, idx_b, isem_b)
        stage(c + 1, idx_b, idx_a, isem_a)
        return carry

    lax.fori_loop(0, NCH // 2, body, 0)
    plsc.subcore_barrier()
    pltpu.sync_copy(acc.at[pl.ds(r0, RPT)], out_hbm.at[cid, pl.ds(r0, RPT)])


_deg_call = pl.kernel(
    _deg_body,
    out_type=jax.ShapeDtypeStruct((2, NP, 16), jnp.float32),
    mesh=_mesh,
    scratch_types=[
        pltpu.VMEM((CH,), jnp.int32),
        pltpu.VMEM((CH,), jnp.int32),
        pltpu.VMEM((CH, 16), jnp.float32),
        pltpu.VMEM_SHARED((NP, 16), jnp.float32),
        pltpu.SemaphoreType.DMA,
        pltpu.SemaphoreType.DMA,
    ],
    compiler_params=_sc_params,
)


def _make_agg(W):
    def body_fn(g_hbm, pk_hbm, zeros_hbm, out_hbm,
                idx_a, idx_b, rows_a, rows_b, tbl, acc,
                isem_a, isem_b, gsem_a, gsem_b):
        cid = lax.axis_index("c")
        sid = lax.axis_index("s")
        r0 = pl.multiple_of(sid * RPT, RPT)
        pltpu.sync_copy(g_hbm.at[pl.ds(r0, RPT)], tbl.at[pl.ds(r0, RPT)])
        pltpu.sync_copy(zeros_hbm.at[pl.ds(r0, RPT)], acc.at[pl.ds(r0, RPT)])
        plsc.subcore_barrier()
        w = cid * NSUB + sid

        def chunk_src(j):
            jj = lax.min(j, NCH - 1)
            return pk_hbm.at[w * NCH + jj]

        pltpu.sync_copy(chunk_src(0), idx_a)
        pltpu.async_copy(tbl.at[idx_a.at[0]], rows_a, gsem_a)

        def stage(c, idx_cur, idx_nxt, rows_cur, rows_nxt, isem_nxt,
                  gsem_cur, gsem_nxt):
            pltpu.async_copy(chunk_src(c + 1), idx_nxt, isem_nxt)
            pltpu.make_async_copy(tbl.at[idx_cur.at[0]], rows_cur,
                                  gsem_cur).wait()
            pltpu.make_async_copy(chunk_src(c + 1), idx_nxt, isem_nxt).wait()
            pltpu.async_copy(tbl.at[idx_nxt.at[0]], rows_nxt, gsem_nxt)
            pltpu.sync_copy(rows_cur, acc.at[idx_cur.at[1]], add=True)

        def body(p, carry):
            c = p * 2
            stage(c, idx_a, idx_b, rows_a, rows_b, isem_b, gsem_a, gsem_b)
            stage(c + 1, idx_b, idx_a, rows_b, rows_a, isem_a, gsem_b, gsem_a)
            return carry

        lax.fori_loop(0, NCH // 2, body, 0)
        pltpu.make_async_copy(tbl.at[idx_a.at[0]], rows_a, gsem_a).wait()
        plsc.subcore_barrier()
        pltpu.sync_copy(acc.at[pl.ds(r0, RPT)], out_hbm.at[cid, pl.ds(r0, RPT)])

    return pl.kernel(
        body_fn,
        out_type=jax.ShapeDtypeStruct((2, NP, W), jnp.float32),
        mesh=_mesh,
        scratch_types=[
            pltpu.VMEM((2, CH), jnp.int32),
            pltpu.VMEM((2, CH), jnp.int32),
            pltpu.VMEM((CH, W), jnp.float32),
            pltpu.VMEM((CH, W), jnp.float32),
            pltpu.VMEM_SHARED((NP, W), jnp.float32),
            pltpu.VMEM_SHARED((NP, W), jnp.float32),
            pltpu.SemaphoreType.DMA,
            pltpu.SemaphoreType.DMA,
            pltpu.SemaphoreType.DMA,
            pltpu.SemaphoreType.DMA,
        ],
        compiler_params=_sc_params,
    )


_agg64 = _make_agg(64)
_agg32 = _make_agg(32)


R = 1024
GRID = NP // R


def _dis(d_ref):
    deg = d_ref[0, :, 0:1] + d_ref[1, :, 0:1] + 1.0
    return lax.rsqrt(deg)


def _l1_body(x_ref, w_ref, d_ref, o_ref):
    h = jnp.dot(x_ref[...], w_ref[...], preferred_element_type=jnp.float32)
    o_ref[...] = h * _dis(d_ref)


def _l1(xp, W1, degp):
    return pl.pallas_call(
        _l1_body,
        grid=(GRID,),
        in_specs=[
            pl.BlockSpec((R, 128), lambda i: (i, 0)),
            pl.BlockSpec((128, 64), lambda i: (0, 0)),
            pl.BlockSpec((2, R, 16), lambda i: (0, i, 0)),
        ],
        out_specs=pl.BlockSpec((R, 64), lambda i: (i, 0)),
        out_shape=jax.ShapeDtypeStruct((NP, 64), jnp.float32),
    )(xp, W1, degp)


def _l2_body(a_ref, g_ref, d_ref, b_ref, w_ref, o_ref):
    dis = _dis(d_ref)
    agg = a_ref[0] + a_ref[1] + g_ref[...]
    h = jnp.maximum(agg * dis + b_ref[...], 0.0)
    o_ref[...] = jnp.dot(h, w_ref[...], preferred_element_type=jnp.float32) * dis


def _l2(a1, g1, degp, b1, W2):
    return pl.pallas_call(
        _l2_body,
        grid=(GRID,),
        in_specs=[
            pl.BlockSpec((2, R, 64), lambda i: (0, i, 0)),
            pl.BlockSpec((R, 64), lambda i: (i, 0)),
            pl.BlockSpec((2, R, 16), lambda i: (0, i, 0)),
            pl.BlockSpec((1, 64), lambda i: (0, 0)),
            pl.BlockSpec((64, 32), lambda i: (0, 0)),
        ],
        out_specs=pl.BlockSpec((R, 32), lambda i: (i, 0)),
        out_shape=jax.ShapeDtypeStruct((NP, 32), jnp.float32),
    )(a1, g1, degp, b1, W2)


def _l3_body(a_ref, g_ref, d_ref, b_ref, o_ref):
    dis = _dis(d_ref)
    agg = a_ref[0] + a_ref[1] + g_ref[...]
    o_ref[...] = jnp.maximum(agg * dis + b_ref[...], 0.0)


def _l3(a2, g2, degp, b2):
    return pl.pallas_call(
        _l3_body,
        grid=(GRID,),
        in_specs=[
            pl.BlockSpec((2, R, 32), lambda i: (0, i, 0)),
            pl.BlockSpec((R, 32), lambda i: (i, 0)),
            pl.BlockSpec((2, R, 16), lambda i: (0, i, 0)),
            pl.BlockSpec((1, 32), lambda i: (0, 0)),
        ],
        out_specs=pl.BlockSpec((R, 32), lambda i: (i, 0)),
        out_shape=jax.ShapeDtypeStruct((NP, 32), jnp.float32),
    )(a2, g2, degp, b2)


def kernel(x, edge_index, W1, b1, W2, b2):
    ei = edge_index.astype(jnp.int32)
    pad_idx = jnp.full((EP - E,), N, jnp.int32)
    src = jnp.concatenate([ei[0], pad_idx])
    dst = jnp.concatenate([ei[1], pad_idx])
    pk = jnp.stack([src, dst], 0).reshape(2, EP // CH, CH).transpose(1, 0, 2)
    xp = jnp.zeros((NP, 128), jnp.float32).at[:N].set(x)
    ones16 = jnp.ones((CH, 16), jnp.float32)
    z16 = jnp.zeros((NP, 16), jnp.float32)
    z64 = jnp.zeros((NP, 64), jnp.float32)
    z32 = jnp.zeros((NP, 32), jnp.float32)

    degp = _deg_call(dst, ones16, z16)
    g1 = _l1(xp, W1, degp)
    a1 = _agg64(g1, pk, z64)
    g2 = _l2(a1, g1, degp, b1.reshape(1, 64), W2)
    a2 = _agg32(g2, pk, z32)
    zp = _l3(a2, g2, degp, b2.reshape(1, 32))
    return zp[:N]


# speedup vs baseline: 29.5104x; 1.0286x over previous
import functools

import jax
import jax.numpy as jnp
from jax import lax
from jax.experimental import pallas as pl
from jax.experimental.pallas import tpu as pltpu
from jax.experimental.pallas import tpu_sc as plsc

N = 10000
E = 320000
NP = 10240
EP = 327680
CH = 128
NW = 32
NCH = EP // (NW * CH)
NSUB = 16
RPT = NP // NSUB

_mesh = plsc.VectorSubcoreMesh(core_axis_name="c", subcore_axis_name="s")
_sc_params = pltpu.CompilerParams(use_tc_tiling_on_sc=False)


def _deg_body(dst_hbm, ones_hbm, zeros_hbm, out_hbm, idx_a, idx_b, ones_v,
              acc, isem_a, isem_b, ssem_a, ssem_b):
    cid = lax.axis_index("c")
    sid = lax.axis_index("s")
    r0 = pl.multiple_of(sid * RPT, RPT)
    pltpu.sync_copy(zeros_hbm.at[pl.ds(r0, RPT)], acc.at[pl.ds(r0, RPT)])
    pltpu.sync_copy(ones_hbm, ones_v)
    plsc.subcore_barrier()
    w = cid * NSUB + sid

    def chunk_src(j):
        jj = lax.min(j, NCH - 1)
        base = pl.multiple_of((w * NCH + jj) * CH, CH)
        return dst_hbm.at[pl.ds(base, CH)]

    pltpu.sync_copy(chunk_src(0), idx_a)
    pltpu.async_copy(ones_v, acc.at[pl.ds(N, CH)], ssem_b)

    def stage(c, idx_cur, idx_nxt, isem_nxt, ssem_cur, ssem_nxt):
        pltpu.async_copy(ones_v, acc.at[idx_cur], add=True, sem=ssem_cur)
        pltpu.make_async_copy(ones_v, acc.at[idx_nxt], ssem_nxt).wait()
        pltpu.async_copy(chunk_src(c + 1), idx_nxt, isem_nxt)
        pltpu.make_async_copy(chunk_src(c + 1), idx_nxt, isem_nxt).wait()

    def body(p, carry):
        c = p * 2
        stage(c, idx_a, idx_b, isem_b, ssem_a, ssem_b)
        stage(c + 1, idx_b, idx_a, isem_a, ssem_b, ssem_a)
        return carry

    lax.fori_loop(0, NCH // 2, body, 0)
    pltpu.make_async_copy(ones_v, acc.at[idx_b], ssem_b).wait()
    plsc.subcore_barrier()
    pltpu.sync_copy(acc.at[pl.ds(r0, RPT)], out_hbm.at[cid, pl.ds(r0, RPT)])


_deg_call = pl.kernel(
    _deg_body,
    out_type=jax.ShapeDtypeStruct((2, NP, 16), jnp.float32),
    mesh=_mesh,
    scratch_types=[
        pltpu.VMEM((CH,), jnp.int32),
        pltpu.VMEM((CH,), jnp.int32),
        pltpu.VMEM((CH, 16), jnp.float32),
        pltpu.VMEM_SHARED((NP, 16), jnp.float32),
        pltpu.SemaphoreType.DMA,
        pltpu.SemaphoreType.DMA,
        pltpu.SemaphoreType.DMA,
        pltpu.SemaphoreType.DMA,
    ],
    compiler_params=_sc_params,
)


def _make_agg(W):
    def body_fn(g_hbm, pk_hbm, zeros_hbm, out_hbm,
                idx_a, idx_b, rows_a, rows_b, tbl, acc,
                isem_a, isem_b, gsem_a, gsem_b, ssem_a, ssem_b):
        cid = lax.axis_index("c")
        sid = lax.axis_index("s")
        r0 = pl.multiple_of(sid * RPT, RPT)
        pltpu.sync_copy(g_hbm.at[pl.ds(r0, RPT)], tbl.at[pl.ds(r0, RPT)])
        pltpu.sync_copy(zeros_hbm.at[pl.ds(r0, RPT)], acc.at[pl.ds(r0, RPT)])
        plsc.subcore_barrier()
        w = cid * NSUB + sid

        def chunk_src(j):
            jj = lax.min(j, NCH - 1)
            return pk_hbm.at[w * NCH + jj]

        pltpu.sync_copy(chunk_src(0), idx_a)
        pltpu.async_copy(tbl.at[idx_a.at[0]], rows_a, gsem_a)
        pltpu.async_copy(rows_b, acc.at[pl.ds(N, CH)], ssem_b)

        def stage(c, idx_cur, idx_nxt, rows_cur, rows_nxt, isem_nxt,
                  gsem_cur, gsem_nxt, ssem_cur, ssem_nxt):
            pltpu.make_async_copy(tbl.at[idx_cur.at[0]], rows_cur,
                                  gsem_cur).wait()
            pltpu.async_copy(rows_cur, acc.at[idx_cur.at[1]], ssem_cur,
                             add=True)
            pltpu.make_async_copy(rows_nxt, acc.at[idx_nxt.at[1]],
                                  ssem_nxt).wait()
            pltpu.async_copy(chunk_src(c + 1), idx_nxt, isem_nxt)
            pltpu.make_async_copy(chunk_src(c + 1), idx_nxt, isem_nxt).wait()
            pltpu.async_copy(tbl.at[idx_nxt.at[0]], rows_nxt, gsem_nxt)

        def body(p, carry):
            c = p * 2
            stage(c, idx_a, idx_b, rows_a, rows_b, isem_b,
                  gsem_a, gsem_b, ssem_a, ssem_b)
            stage(c + 1, idx_b, idx_a, rows_b, rows_a, isem_a,
                  gsem_b, gsem_a, ssem_b, ssem_a)
            return carry

        lax.fori_loop(0, NCH // 2, body, 0)
        pltpu.make_async_copy(tbl.at[idx_a.at[0]], rows_a, gsem_a).wait()
        pltpu.make_async_copy(rows_b, acc.at[idx_b.at[1]], ssem_b).wait()
        plsc.subcore_barrier()
        pltpu.sync_copy(acc.at[pl.ds(r0, RPT)], out_hbm.at[cid, pl.ds(r0, RPT)])

    return pl.kernel(
        body_fn,
        out_type=jax.ShapeDtypeStruct((2, NP, W), jnp.float32),
        mesh=_mesh,
        scratch_types=[
            pltpu.VMEM((2, CH), jnp.int32),
            pltpu.VMEM((2, CH), jnp.int32),
            pltpu.VMEM((CH, W), jnp.float32),
            pltpu.VMEM((CH, W), jnp.float32),
            pltpu.VMEM_SHARED((NP, W), jnp.float32),
            pltpu.VMEM_SHARED((NP, W), jnp.float32),
            pltpu.SemaphoreType.DMA,
            pltpu.SemaphoreType.DMA,
            pltpu.SemaphoreType.DMA,
            pltpu.SemaphoreType.DMA,
            pltpu.SemaphoreType.DMA,
            pltpu.SemaphoreType.DMA,
        ],
        compiler_params=_sc_params,
    )


_agg64 = _make_agg(64)
_agg32 = _make_agg(32)


R = 1024
GRID = NP // R


def _dis(d_ref):
    deg = d_ref[0, :, 0:1] + d_ref[1, :, 0:1] + 1.0
    return lax.rsqrt(deg)


def _l1_body(x_ref, w_ref, d_ref, o_ref):
    h = jnp.dot(x_ref[...], w_ref[...], preferred_element_type=jnp.float32)
    o_ref[...] = h * _dis(d_ref)


def _l1(xp, W1, degp):
    return pl.pallas_call(
        _l1_body,
        grid=(GRID,),
        in_specs=[
            pl.BlockSpec((R, 128), lambda i: (i, 0)),
            pl.BlockSpec((128, 64), lambda i: (0, 0)),
            pl.BlockSpec((2, R, 16), lambda i: (0, i, 0)),
        ],
        out_specs=pl.BlockSpec((R, 64), lambda i: (i, 0)),
        out_shape=jax.ShapeDtypeStruct((NP, 64), jnp.float32),
    )(xp, W1, degp)


def _l2_body(a_ref, g_ref, d_ref, b_ref, w_ref, o_ref):
    dis = _dis(d_ref)
    agg = a_ref[0] + a_ref[1] + g_ref[...]
    h = jnp.maximum(agg * dis + b_ref[...], 0.0)
    o_ref[...] = jnp.dot(h, w_ref[...], preferred_element_type=jnp.float32) * dis


def _l2(a1, g1, degp, b1, W2):
    return pl.pallas_call(
        _l2_body,
        grid=(GRID,),
        in_specs=[
            pl.BlockSpec((2, R, 64), lambda i: (0, i, 0)),
            pl.BlockSpec((R, 64), lambda i: (i, 0)),
            pl.BlockSpec((2, R, 16), lambda i: (0, i, 0)),
            pl.BlockSpec((1, 64), lambda i: (0, 0)),
            pl.BlockSpec((64, 32), lambda i: (0, 0)),
        ],
        out_specs=pl.BlockSpec((R, 32), lambda i: (i, 0)),
        out_shape=jax.ShapeDtypeStruct((NP, 32), jnp.float32),
    )(a1, g1, degp, b1, W2)


def _l3_body(a_ref, g_ref, d_ref, b_ref, o_ref):
    dis = _dis(d_ref)
    agg = a_ref[0] + a_ref[1] + g_ref[...]
    o_ref[...] = jnp.maximum(agg * dis + b_ref[...], 0.0)


def _l3(a2, g2, degp, b2):
    return pl.pallas_call(
        _l3_body,
        grid=(GRID,),
        in_specs=[
            pl.BlockSpec((2, R, 32), lambda i: (0, i, 0)),
            pl.BlockSpec((R, 32), lambda i: (i, 0)),
            pl.BlockSpec((2, R, 16), lambda i: (0, i, 0)),
            pl.BlockSpec((1, 32), lambda i: (0, 0)),
        ],
        out_specs=pl.BlockSpec((R, 32), lambda i: (i, 0)),
        out_shape=jax.ShapeDtypeStruct((NP, 32), jnp.float32),
    )(a2, g2, degp, b2)


def kernel(x, edge_index, W1, b1, W2, b2):
    ei = edge_index.astype(jnp.int32)
    pad_idx = jnp.full((EP - E,), N, jnp.int32)
    src = jnp.concatenate([ei[0], pad_idx])
    dst = jnp.concatenate([ei[1], pad_idx])
    pk = jnp.stack([src, dst], 0).reshape(2, EP // CH, CH).transpose(1, 0, 2)
    xp = jnp.zeros((NP, 128), jnp.float32).at[:N].set(x)
    ones16 = jnp.ones((CH, 16), jnp.float32)
    z16 = jnp.zeros((NP, 16), jnp.float32)
    z64 = jnp.zeros((NP, 64), jnp.float32)
    z32 = jnp.zeros((NP, 32), jnp.float32)

    degp = _deg_call(dst, ones16, z16)
    g1 = _l1(xp, W1, degp)
    a1 = _agg64(g1, pk, z64)
    g2 = _l2(a1, g1, degp, b1.reshape(1, 64), W2)
    a2 = _agg32(g2, pk, z32)
    zp = _l3(a2, g2, degp, b2.reshape(1, 32))
    return zp[:N]


# speedup vs baseline: 37.9204x; 1.2850x over previous
import functools

import jax
import jax.numpy as jnp
from jax import lax
from jax.experimental import pallas as pl
from jax.experimental.pallas import tpu as pltpu
from jax.experimental.pallas import tpu_sc as plsc

N = 10000
E = 320000
NP = 10240
EP = 327680
CH = 128
NW = 32
NCH = EP // (NW * CH)
KB = 8
NBLK = NCH // KB
NSUB = 16
RPT = NP // NSUB

_mesh = plsc.VectorSubcoreMesh(core_axis_name="c", subcore_axis_name="s")
_sc_params = pltpu.CompilerParams(use_tc_tiling_on_sc=False)


def _deg_body(dstr_hbm, ones_hbm, zeros_hbm, out_hbm, idx_a, idx_b, ones_v,
              acc, isem_a, isem_b, ssem_a, ssem_b):
    cid = lax.axis_index("c")
    sid = lax.axis_index("s")
    r0 = pl.multiple_of(sid * RPT, RPT)
    pltpu.sync_copy(zeros_hbm.at[pl.ds(r0, RPT)], acc.at[pl.ds(r0, RPT)])
    pltpu.sync_copy(ones_hbm, ones_v)
    plsc.subcore_barrier()
    w = cid * NSUB + sid

    def blk_src(b):
        bb = lax.min(b, NBLK - 1)
        return dstr_hbm.at[pl.ds(pl.multiple_of((w * NBLK + bb) * KB, KB), KB)]

    pltpu.async_copy(blk_src(0), idx_a, isem_a)
    pltpu.async_copy(ones_v, acc.at[pl.ds(N, CH)], ssem_b)

    def block(b, idx_cur, idx_nxt, isem_cur, isem_nxt):
        pltpu.make_async_copy(blk_src(b), idx_cur, isem_cur).wait()
        pltpu.async_copy(blk_src(b + 1), idx_nxt, isem_nxt)
        for kk in range(KB):
            sc, sn = (ssem_a, ssem_b) if kk % 2 == 0 else (ssem_b, ssem_a)
            pltpu.async_copy(ones_v, acc.at[idx_cur.at[kk]], add=True, sem=sc)
            pltpu.make_async_copy(ones_v, acc.at[idx_cur.at[kk]], sn).wait()

    def body(p, carry):
        block(p * 2, idx_a, idx_b, isem_a, isem_b)
        block(p * 2 + 1, idx_b, idx_a, isem_b, isem_a)
        return carry

    lax.fori_loop(0, NBLK // 2, body, 0)
    pltpu.make_async_copy(ones_v, acc.at[idx_b.at[KB - 1]], ssem_b).wait()
    pltpu.make_async_copy(blk_src(NBLK - 1), idx_a, isem_a).wait()
    plsc.subcore_barrier()
    pltpu.sync_copy(acc.at[pl.ds(r0, RPT)], out_hbm.at[cid, pl.ds(r0, RPT)])


_deg_call = pl.kernel(
    _deg_body,
    out_type=jax.ShapeDtypeStruct((2, NP, 16), jnp.float32),
    mesh=_mesh,
    scratch_types=[
        pltpu.VMEM((KB, CH), jnp.int32),
        pltpu.VMEM((KB, CH), jnp.int32),
        pltpu.VMEM((CH, 16), jnp.float32),
        pltpu.VMEM_SHARED((NP, 16), jnp.float32),
        pltpu.SemaphoreType.DMA,
        pltpu.SemaphoreType.DMA,
        pltpu.SemaphoreType.DMA,
        pltpu.SemaphoreType.DMA,
    ],
    compiler_params=_sc_params,
)


def _make_agg(W):
    def body_fn(g_hbm, pk_hbm, zeros_hbm, out_hbm,
                idx_a, idx_b, rows_a, rows_b, tbl, acc,
                isem_a, isem_b, gsem_a, gsem_b, ssem_a, ssem_b):
        cid = lax.axis_index("c")
        sid = lax.axis_index("s")
        r0 = pl.multiple_of(sid * RPT, RPT)
        pltpu.sync_copy(g_hbm.at[pl.ds(r0, RPT)], tbl.at[pl.ds(r0, RPT)])
        pltpu.sync_copy(zeros_hbm.at[pl.ds(r0, RPT)], acc.at[pl.ds(r0, RPT)])
        plsc.subcore_barrier()
        w = cid * NSUB + sid

        def blk_src(b):
            bb = lax.min(b, NBLK - 1)
            return pk_hbm.at[pl.ds(pl.multiple_of((w * NBLK + bb) * KB, KB),
                                   KB)]

        pltpu.sync_copy(blk_src(0), idx_a)
        pltpu.async_copy(tbl.at[idx_a.at[0, 0]], rows_a, gsem_a)
        pltpu.async_copy(rows_b, acc.at[pl.ds(N, CH)], ssem_b)

        def stage(icur, inxt, rows_cur, rows_nxt,
                  gsem_cur, gsem_nxt, ssem_cur, ssem_nxt):
            pltpu.make_async_copy(tbl.at[icur.at[0]], rows_cur,
                                  gsem_cur).wait()
            pltpu.async_copy(rows_cur, acc.at[icur.at[1]], ssem_cur,
                             add=True)
            pltpu.make_async_copy(rows_nxt, acc.at[icur.at[1]],
                                  ssem_nxt).wait()
            pltpu.async_copy(tbl.at[inxt.at[0]], rows_nxt, gsem_nxt)

        def block(b, idx_cur, idx_nxt, isem_nxt):
            pltpu.async_copy(blk_src(b + 1), idx_nxt, isem_nxt)
            for kk in range(KB):
                sems = ((gsem_a, gsem_b, ssem_a, ssem_b) if kk % 2 == 0
                        else (gsem_b, gsem_a, ssem_b, ssem_a))
                if kk < KB - 1:
                    icur, inxt = idx_cur.at[kk], idx_cur.at[kk + 1]
                else:
                    pltpu.make_async_copy(blk_src(b + 1), idx_nxt,
                                          isem_nxt).wait()
                    icur, inxt = idx_cur.at[kk], idx_nxt.at[0]
                rc, rn = (rows_a, rows_b) if kk % 2 == 0 else (rows_b, rows_a)
                stage(icur, inxt, rc, rn, *sems)

        def body(p, carry):
            block(p * 2, idx_a, idx_b, isem_b)
            block(p * 2 + 1, idx_b, idx_a, isem_a)
            return carry

        lax.fori_loop(0, NBLK // 2, body, 0)
        pltpu.make_async_copy(tbl.at[idx_a.at[0, 0]], rows_a, gsem_a).wait()
        pltpu.make_async_copy(rows_b, acc.at[idx_b.at[KB - 1, 1]],
                              ssem_b).wait()
        plsc.subcore_barrier()
        pltpu.sync_copy(acc.at[pl.ds(r0, RPT)], out_hbm.at[cid, pl.ds(r0, RPT)])

    return pl.kernel(
        body_fn,
        out_type=jax.ShapeDtypeStruct((2, NP, W), jnp.float32),
        mesh=_mesh,
        scratch_types=[
            pltpu.VMEM((KB, 2, CH), jnp.int32),
            pltpu.VMEM((KB, 2, CH), jnp.int32),
            pltpu.VMEM((CH, W), jnp.float32),
            pltpu.VMEM((CH, W), jnp.float32),
            pltpu.VMEM_SHARED((NP, W), jnp.float32),
            pltpu.VMEM_SHARED((NP, W), jnp.float32),
            pltpu.SemaphoreType.DMA,
            pltpu.SemaphoreType.DMA,
            pltpu.SemaphoreType.DMA,
            pltpu.SemaphoreType.DMA,
            pltpu.SemaphoreType.DMA,
            pltpu.SemaphoreType.DMA,
        ],
        compiler_params=_sc_params,
    )


_agg64 = _make_agg(64)
_agg32 = _make_agg(32)


R = 1024
GRID = NP // R


def _dis(d_ref):
    deg = d_ref[0, :, 0:1] + d_ref[1, :, 0:1] + 1.0
    return lax.rsqrt(deg)


def _l1_body(x_ref, w_ref, d_ref, o_ref):
    h = jnp.dot(x_ref[...], w_ref[...], preferred_element_type=jnp.float32)
    o_ref[...] = h * _dis(d_ref)


def _l1(xp, W1, degp):
    return pl.pallas_call(
        _l1_body,
        grid=(GRID,),
        in_specs=[
            pl.BlockSpec((R, 128), lambda i: (i, 0)),
            pl.BlockSpec((128, 64), lambda i: (0, 0)),
            pl.BlockSpec((2, R, 16), lambda i: (0, i, 0)),
        ],
        out_specs=pl.BlockSpec((R, 64), lambda i: (i, 0)),
        out_shape=jax.ShapeDtypeStruct((NP, 64), jnp.float32),
    )(xp, W1, degp)


def _l2_body(a_ref, g_ref, d_ref, b_ref, w_ref, o_ref):
    dis = _dis(d_ref)
    agg = a_ref[0] + a_ref[1] + g_ref[...]
    h = jnp.maximum(agg * dis + b_ref[...], 0.0)
    o_ref[...] = jnp.dot(h, w_ref[...], preferred_element_type=jnp.float32) * dis


def _l2(a1, g1, degp, b1, W2):
    return pl.pallas_call(
        _l2_body,
        grid=(GRID,),
        in_specs=[
            pl.BlockSpec((2, R, 64), lambda i: (0, i, 0)),
            pl.BlockSpec((R, 64), lambda i: (i, 0)),
            pl.BlockSpec((2, R, 16), lambda i: (0, i, 0)),
            pl.BlockSpec((1, 64), lambda i: (0, 0)),
            pl.BlockSpec((64, 32), lambda i: (0, 0)),
        ],
        out_specs=pl.BlockSpec((R, 32), lambda i: (i, 0)),
        out_shape=jax.ShapeDtypeStruct((NP, 32), jnp.float32),
    )(a1, g1, degp, b1, W2)


def _l3_body(a_ref, g_ref, d_ref, b_ref, o_ref):
    dis = _dis(d_ref)
    agg = a_ref[0] + a_ref[1] + g_ref[...]
    o_ref[...] = jnp.maximum(agg * dis + b_ref[...], 0.0)


def _l3(a2, g2, degp, b2):
    return pl.pallas_call(
        _l3_body,
        grid=(GRID,),
        in_specs=[
            pl.BlockSpec((2, R, 32), lambda i: (0, i, 0)),
            pl.BlockSpec((R, 32), lambda i: (i, 0)),
            pl.BlockSpec((2, R, 16), lambda i: (0, i, 0)),
            pl.BlockSpec((1, 32), lambda i: (0, 0)),
        ],
        out_specs=pl.BlockSpec((R, 32), lambda i: (i, 0)),
        out_shape=jax.ShapeDtypeStruct((NP, 32), jnp.float32),
    )(a2, g2, degp, b2)


def kernel(x, edge_index, W1, b1, W2, b2):
    ei = edge_index.astype(jnp.int32)
    pad_idx = jnp.full((EP - E,), N, jnp.int32)
    src = jnp.concatenate([ei[0], pad_idx])
    dst = jnp.concatenate([ei[1], pad_idx])
    pk = jnp.stack([src, dst], 0).reshape(2, EP // CH, CH).transpose(1, 0, 2)
    xp = jnp.zeros((NP, 128), jnp.float32).at[:N].set(x)
    ones16 = jnp.ones((CH, 16), jnp.float32)
    z16 = jnp.zeros((NP, 16), jnp.float32)
    z64 = jnp.zeros((NP, 64), jnp.float32)
    z32 = jnp.zeros((NP, 32), jnp.float32)

    degp = _deg_call(dst.reshape(EP // CH, CH), ones16, z16)
    g1 = _l1(xp, W1, degp)
    a1 = _agg64(g1, pk, z64)
    g2 = _l2(a1, g1, degp, b1.reshape(1, 64), W2)
    a2 = _agg32(g2, pk, z32)
    zp = _l3(a2, g2, degp, b2.reshape(1, 32))
    return zp[:N]


# speedup vs baseline: 38.7117x; 1.0209x over previous
import functools

import jax
import jax.numpy as jnp
from jax import lax
from jax.experimental import pallas as pl
from jax.experimental.pallas import tpu as pltpu
from jax.experimental.pallas import tpu_sc as plsc

N = 10000
E = 320000
NP = 10240
EP = 327680
CH = 128
NW = 32
NCH = EP // (NW * CH)
KB = 8
NBLK = NCH // KB
NSUB = 16
RPT = NP // NSUB

_mesh = plsc.VectorSubcoreMesh(core_axis_name="c", subcore_axis_name="s")
_sc_params = pltpu.CompilerParams(use_tc_tiling_on_sc=False)


def _deg_body(dstr_hbm, ones_hbm, zeros_hbm, out_hbm, idx_a, idx_b, ones_v,
              acc, isem_a, isem_b, ssem_a, ssem_b):
    cid = lax.axis_index("c")
    sid = lax.axis_index("s")
    r0 = pl.multiple_of(sid * RPT, RPT)
    pltpu.sync_copy(zeros_hbm.at[pl.ds(r0, RPT)], acc.at[pl.ds(r0, RPT)])
    pltpu.sync_copy(ones_hbm, ones_v)
    plsc.subcore_barrier()
    w = cid * NSUB + sid

    def blk_src(b):
        bb = lax.min(b, NBLK - 1)
        return dstr_hbm.at[pl.ds(pl.multiple_of((w * NBLK + bb) * KB, KB), KB)]

    pltpu.async_copy(blk_src(0), idx_a, isem_a)
    pltpu.async_copy(ones_v, acc.at[pl.ds(N, CH)], ssem_b)

    def block(b, idx_cur, idx_nxt, isem_cur, isem_nxt):
        pltpu.make_async_copy(blk_src(b), idx_cur, isem_cur).wait()
        pltpu.async_copy(blk_src(b + 1), idx_nxt, isem_nxt)
        for kk in range(KB):
            sc, sn = (ssem_a, ssem_b) if kk % 2 == 0 else (ssem_b, ssem_a)
            pltpu.async_copy(ones_v, acc.at[idx_cur.at[kk]], add=True, sem=sc)
            pltpu.make_async_copy(ones_v, acc.at[idx_cur.at[kk]], sn).wait()

    def body(p, carry):
        block(p * 2, idx_a, idx_b, isem_a, isem_b)
        block(p * 2 + 1, idx_b, idx_a, isem_b, isem_a)
        return carry

    lax.fori_loop(0, NBLK // 2, body, 0)
    pltpu.make_async_copy(ones_v, acc.at[idx_b.at[KB - 1]], ssem_b).wait()
    pltpu.make_async_copy(blk_src(NBLK - 1), idx_a, isem_a).wait()
    plsc.subcore_barrier()
    pltpu.sync_copy(acc.at[pl.ds(r0, RPT)], out_hbm.at[cid, pl.ds(r0, RPT)])


_deg_call = pl.kernel(
    _deg_body,
    out_type=jax.ShapeDtypeStruct((2, NP, 16), jnp.float32),
    mesh=_mesh,
    scratch_types=[
        pltpu.VMEM((KB, CH), jnp.int32),
        pltpu.VMEM((KB, CH), jnp.int32),
        pltpu.VMEM((CH, 16), jnp.float32),
        pltpu.VMEM_SHARED((NP, 16), jnp.float32),
        pltpu.SemaphoreType.DMA,
        pltpu.SemaphoreType.DMA,
        pltpu.SemaphoreType.DMA,
        pltpu.SemaphoreType.DMA,
    ],
    compiler_params=_sc_params,
)


def _make_agg(W):
    def body_fn(g_hbm, srcr_hbm, dstr_hbm, zeros_hbm, out_hbm,
                sidx_a, sidx_b, didx_a, didx_b, rows_a, rows_b, tbl, acc,
                isem_a, isem_b, gsem_a, gsem_b, ssem_a, ssem_b):
        cid = lax.axis_index("c")
        sid = lax.axis_index("s")
        r0 = pl.multiple_of(sid * RPT, RPT)
        pltpu.sync_copy(g_hbm.at[pl.ds(r0, RPT)], tbl.at[pl.ds(r0, RPT)])
        pltpu.sync_copy(zeros_hbm.at[pl.ds(r0, RPT)], acc.at[pl.ds(r0, RPT)])
        plsc.subcore_barrier()
        w = cid * NSUB + sid

        def blk(ref, b):
            bb = lax.min(b, NBLK - 1)
            return ref.at[pl.ds(pl.multiple_of((w * NBLK + bb) * KB, KB), KB)]

        pltpu.sync_copy(blk(srcr_hbm, 0), sidx_a)
        pltpu.sync_copy(blk(dstr_hbm, 0), didx_a)
        pltpu.async_copy(tbl.at[sidx_a.at[0]], rows_a, gsem_a)
        pltpu.async_copy(rows_b, acc.at[pl.ds(N, CH)], ssem_b)

        def stage(scur, dcur, snxt, rows_cur, rows_nxt,
                  gsem_cur, gsem_nxt, ssem_cur, ssem_nxt):
            pltpu.make_async_copy(tbl.at[scur], rows_cur, gsem_cur).wait()
            pltpu.async_copy(rows_cur, acc.at[dcur], ssem_cur, add=True)
            pltpu.make_async_copy(rows_nxt, acc.at[dcur], ssem_nxt).wait()
            pltpu.async_copy(tbl.at[snxt], rows_nxt, gsem_nxt)

        def block(b, scur_b, snxt_b, dcur_b, dnxt_b, isem_nxt):
            pltpu.async_copy(blk(srcr_hbm, b + 1), snxt_b, isem_nxt)
            pltpu.async_copy(blk(dstr_hbm, b + 1), dnxt_b, isem_nxt)
            for kk in range(KB):
                sems = ((gsem_a, gsem_b, ssem_a, ssem_b) if kk % 2 == 0
                        else (gsem_b, gsem_a, ssem_b, ssem_a))
                if kk < KB - 1:
                    scur, snxt = scur_b.at[kk], scur_b.at[kk + 1]
                else:
                    pltpu.make_async_copy(blk(srcr_hbm, b + 1), snxt_b,
                                          isem_nxt).wait()
                    pltpu.make_async_copy(blk(dstr_hbm, b + 1), dnxt_b,
                                          isem_nxt).wait()
                    scur, snxt = scur_b.at[kk], snxt_b.at[0]
                rc, rn = (rows_a, rows_b) if kk % 2 == 0 else (rows_b, rows_a)
                stage(scur, dcur_b.at[kk], snxt, rc, rn, *sems)

        def body(p, carry):
            block(p * 2, sidx_a, sidx_b, didx_a, didx_b, isem_b)
            block(p * 2 + 1, sidx_b, sidx_a, didx_b, didx_a, isem_a)
            return carry

        lax.fori_loop(0, NBLK // 2, body, 0)
        pltpu.make_async_copy(tbl.at[sidx_a.at[0]], rows_a, gsem_a).wait()
        pltpu.make_async_copy(rows_b, acc.at[didx_b.at[KB - 1]],
                              ssem_b).wait()
        plsc.subcore_barrier()
        pltpu.sync_copy(acc.at[pl.ds(r0, RPT)], out_hbm.at[cid, pl.ds(r0, RPT)])

    return pl.kernel(
        body_fn,
        out_type=jax.ShapeDtypeStruct((2, NP, W), jnp.float32),
        mesh=_mesh,
        scratch_types=[
            pltpu.VMEM((KB, CH), jnp.int32),
            pltpu.VMEM((KB, CH), jnp.int32),
            pltpu.VMEM((KB, CH), jnp.int32),
            pltpu.VMEM((KB, CH), jnp.int32),
            pltpu.VMEM((CH, W), jnp.float32),
            pltpu.VMEM((CH, W), jnp.float32),
            pltpu.VMEM_SHARED((NP, W), jnp.float32),
            pltpu.VMEM_SHARED((NP, W), jnp.float32),
            pltpu.SemaphoreType.DMA,
            pltpu.SemaphoreType.DMA,
            pltpu.SemaphoreType.DMA,
            pltpu.SemaphoreType.DMA,
            pltpu.SemaphoreType.DMA,
            pltpu.SemaphoreType.DMA,
        ],
        compiler_params=_sc_params,
    )


_agg64 = _make_agg(64)
_agg32 = _make_agg(32)


R = 2048
GRID = NP // R
R3 = 2000
GRID3 = N // R3


def _dis(d_ref):
    deg = d_ref[0, :, 0:1] + d_ref[1, :, 0:1] + 1.0
    return lax.rsqrt(deg)


def _l1_body(x_ref, w_ref, d_ref, o_ref):
    h = jnp.dot(x_ref[...], w_ref[...], preferred_element_type=jnp.float32)
    o_ref[...] = h * _dis(d_ref)


def _l1(xp, W1, degp):
    return pl.pallas_call(
        _l1_body,
        grid=(GRID,),
        in_specs=[
            pl.BlockSpec((R, 128), lambda i: (i, 0)),
            pl.BlockSpec((128, 64), lambda i: (0, 0)),
            pl.BlockSpec((2, R, 16), lambda i: (0, i, 0)),
        ],
        out_specs=pl.BlockSpec((R, 64), lambda i: (i, 0)),
        out_shape=jax.ShapeDtypeStruct((NP, 64), jnp.float32),
    )(xp, W1, degp)


def _l2_body(a_ref, g_ref, d_ref, b_ref, w_ref, o_ref):
    dis = _dis(d_ref)
    agg = a_ref[0] + a_ref[1] + g_ref[...]
    h = jnp.maximum(agg * dis + b_ref[...], 0.0)
    o_ref[...] = jnp.dot(h, w_ref[...], preferred_element_type=jnp.float32) * dis


def _l2(a1, g1, degp, b1, W2):
    return pl.pallas_call(
        _l2_body,
        grid=(GRID,),
        in_specs=[
            pl.BlockSpec((2, R, 64), lambda i: (0, i, 0)),
            pl.BlockSpec((R, 64), lambda i: (i, 0)),
            pl.BlockSpec((2, R, 16), lambda i: (0, i, 0)),
            pl.BlockSpec((1, 64), lambda i: (0, 0)),
            pl.BlockSpec((64, 32), lambda i: (0, 0)),
        ],
        out_specs=pl.BlockSpec((R, 32), lambda i: (i, 0)),
        out_shape=jax.ShapeDtypeStruct((NP, 32), jnp.float32),
    )(a1, g1, degp, b1, W2)


def _l3_body(a_ref, g_ref, d_ref, b_ref, o_ref):
    dis = _dis(d_ref)
    agg = a_ref[0] + a_ref[1] + g_ref[...]
    o_ref[...] = jnp.maximum(agg * dis + b_ref[...], 0.0)


def _l3(a2, g2, degp, b2):
    return pl.pallas_call(
        _l3_body,
        grid=(GRID3,),
        in_specs=[
            pl.BlockSpec((2, R3, 32), lambda i: (0, i, 0)),
            pl.BlockSpec((R3, 32), lambda i: (i, 0)),
            pl.BlockSpec((2, R3, 16), lambda i: (0, i, 0)),
            pl.BlockSpec((1, 32), lambda i: (0, 0)),
        ],
        out_specs=pl.BlockSpec((R3, 32), lambda i: (i, 0)),
        out_shape=jax.ShapeDtypeStruct((N, 32), jnp.float32),
    )(a2, g2, degp, b2)


def kernel(x, edge_index, W1, b1, W2, b2):
    ei = edge_index.astype(jnp.int32)
    pad_idx = jnp.full((EP - E,), N, jnp.int32)
    srcr = jnp.concatenate([ei[0], pad_idx]).reshape(EP // CH, CH)
    dstr = jnp.concatenate([ei[1], pad_idx]).reshape(EP // CH, CH)
    xp = jnp.zeros((NP, 128), jnp.float32).at[:N].set(x)
    ones16 = jnp.ones((CH, 16), jnp.float32)
    z16 = jnp.zeros((NP, 16), jnp.float32)
    z64 = jnp.zeros((NP, 64), jnp.float32)
    z32 = jnp.zeros((NP, 32), jnp.float32)

    degp = _deg_call(dstr, ones16, z16)
    g1 = _l1(xp, W1, degp)
    a1 = _agg64(g1, srcr, dstr, z64)
    g2 = _l2(a1, g1, degp, b1.reshape(1, 64), W2)
    a2 = _agg32(g2, srcr, dstr, z32)
    return _l3(a2, g2, degp, b2.reshape(1, 32))


# speedup vs baseline: 40.6923x; 1.0512x over previous
import functools

import jax
import jax.numpy as jnp
from jax import lax
from jax.experimental import pallas as pl
from jax.experimental.pallas import tpu as pltpu
from jax.experimental.pallas import tpu_sc as plsc

N = 10000
E = 320000
NP = 10240
EP = 327680
CH = 128
NW = 32
NCH = EP // (NW * CH)
KB = 8
NBLK = NCH // KB
NSUB = 16
RPT = NP // NSUB

_mesh = plsc.VectorSubcoreMesh(core_axis_name="c", subcore_axis_name="s")
_sc_params = pltpu.CompilerParams(use_tc_tiling_on_sc=False)


def _deg_body(eir_hbm, ones_hbm, zeros_hbm, out_hbm, idx_a, idx_b, ones_v,
              acc, isem_a, isem_b, ssem_a, ssem_b):
    dstr_hbm = eir_hbm.at[1]
    cid = lax.axis_index("c")
    sid = lax.axis_index("s")
    r0 = pl.multiple_of(sid * RPT, RPT)
    pltpu.sync_copy(zeros_hbm.at[pl.ds(r0, RPT)], acc.at[pl.ds(r0, RPT)])
    pltpu.sync_copy(ones_hbm, ones_v)
    plsc.subcore_barrier()
    w = cid * NSUB + sid

    def blk_src(b):
        bb = lax.min(b, NBLK - 1)
        return dstr_hbm.at[pl.ds(pl.multiple_of((w * NBLK + bb) * KB, KB), KB)]

    pltpu.async_copy(blk_src(0), idx_a, isem_a)
    pltpu.async_copy(ones_v, acc.at[pl.ds(N, CH)], ssem_b)

    def block(b, idx_cur, idx_nxt, isem_cur, isem_nxt):
        pltpu.make_async_copy(blk_src(b), idx_cur, isem_cur).wait()
        pltpu.async_copy(blk_src(b + 1), idx_nxt, isem_nxt)
        for kk in range(KB):
            sc, sn = (ssem_a, ssem_b) if kk % 2 == 0 else (ssem_b, ssem_a)
            pltpu.async_copy(ones_v, acc.at[idx_cur.at[kk]], add=True, sem=sc)
            pltpu.make_async_copy(ones_v, acc.at[idx_cur.at[kk]], sn).wait()

    def body(p, carry):
        block(p * 2, idx_a, idx_b, isem_a, isem_b)
        block(p * 2 + 1, idx_b, idx_a, isem_b, isem_a)
        return carry

    lax.fori_loop(0, NBLK // 2, body, 0)
    pltpu.make_async_copy(ones_v, acc.at[idx_b.at[KB - 1]], ssem_b).wait()
    pltpu.make_async_copy(blk_src(NBLK - 1), idx_a, isem_a).wait()
    plsc.subcore_barrier()
    pltpu.sync_copy(acc.at[pl.ds(r0, RPT)], out_hbm.at[cid, pl.ds(r0, RPT)])


_deg_call = pl.kernel(
    _deg_body,
    out_type=jax.ShapeDtypeStruct((2, NP, 16), jnp.float32),
    mesh=_mesh,
    scratch_types=[
        pltpu.VMEM((KB, CH), jnp.int32),
        pltpu.VMEM((KB, CH), jnp.int32),
        pltpu.VMEM((CH, 16), jnp.float32),
        pltpu.VMEM_SHARED((NP, 16), jnp.float32),
        pltpu.SemaphoreType.DMA,
        pltpu.SemaphoreType.DMA,
        pltpu.SemaphoreType.DMA,
        pltpu.SemaphoreType.DMA,
    ],
    compiler_params=_sc_params,
)


def _make_agg(W):
    def body_fn(g_hbm, eir_hbm, zeros_hbm, out_hbm,
                sidx_a, sidx_b, didx_a, didx_b, rows_a, rows_b, tbl, acc,
                isem_a, isem_b, gsem_a, gsem_b, ssem_a, ssem_b):
        cid = lax.axis_index("c")
        sid = lax.axis_index("s")
        r0 = pl.multiple_of(sid * RPT, RPT)
        pltpu.sync_copy(g_hbm.at[pl.ds(r0, RPT)], tbl.at[pl.ds(r0, RPT)])
        pltpu.sync_copy(zeros_hbm.at[pl.ds(r0, RPT)], acc.at[pl.ds(r0, RPT)])
        plsc.subcore_barrier()
        w = cid * NSUB + sid

        srcr_hbm = eir_hbm.at[0]
        dstr_hbm = eir_hbm.at[1]

        def blk(ref, b):
            bb = lax.min(b, NBLK - 1)
            return ref.at[pl.ds(pl.multiple_of((w * NBLK + bb) * KB, KB), KB)]

        pltpu.sync_copy(blk(srcr_hbm, 0), sidx_a)
        pltpu.sync_copy(blk(dstr_hbm, 0), didx_a)
        pltpu.async_copy(tbl.at[sidx_a.at[0]], rows_a, gsem_a)
        pltpu.async_copy(rows_b, acc.at[pl.ds(N, CH)], ssem_b)

        def stage(scur, dcur, snxt, rows_cur, rows_nxt,
                  gsem_cur, gsem_nxt, ssem_cur, ssem_nxt):
            pltpu.make_async_copy(tbl.at[scur], rows_cur, gsem_cur).wait()
            pltpu.async_copy(rows_cur, acc.at[dcur], ssem_cur, add=True)
            pltpu.make_async_copy(rows_nxt, acc.at[dcur], ssem_nxt).wait()
            pltpu.async_copy(tbl.at[snxt], rows_nxt, gsem_nxt)

        def block(b, scur_b, snxt_b, dcur_b, dnxt_b, isem_nxt):
            pltpu.async_copy(blk(srcr_hbm, b + 1), snxt_b, isem_nxt)
            pltpu.async_copy(blk(dstr_hbm, b + 1), dnxt_b, isem_nxt)
            for kk in range(KB):
                sems = ((gsem_a, gsem_b, ssem_a, ssem_b) if kk % 2 == 0
                        else (gsem_b, gsem_a, ssem_b, ssem_a))
                if kk < KB - 1:
                    scur, snxt = scur_b.at[kk], scur_b.at[kk + 1]
                else:
                    pltpu.make_async_copy(blk(srcr_hbm, b + 1), snxt_b,
                                          isem_nxt).wait()
                    pltpu.make_async_copy(blk(dstr_hbm, b + 1), dnxt_b,
                                          isem_nxt).wait()
                    scur, snxt = scur_b.at[kk], snxt_b.at[0]
                rc, rn = (rows_a, rows_b) if kk % 2 == 0 else (rows_b, rows_a)
                stage(scur, dcur_b.at[kk], snxt, rc, rn, *sems)

        def body(p, carry):
            block(p * 2, sidx_a, sidx_b, didx_a, didx_b, isem_b)
            block(p * 2 + 1, sidx_b, sidx_a, didx_b, didx_a, isem_a)
            return carry

        lax.fori_loop(0, NBLK // 2, body, 0)
        pltpu.make_async_copy(tbl.at[sidx_a.at[0]], rows_a, gsem_a).wait()
        pltpu.make_async_copy(rows_b, acc.at[didx_b.at[KB - 1]],
                              ssem_b).wait()
        plsc.subcore_barrier()
        pltpu.sync_copy(acc.at[pl.ds(r0, RPT)], out_hbm.at[cid, pl.ds(r0, RPT)])

    return pl.kernel(
        body_fn,
        out_type=jax.ShapeDtypeStruct((2, NP, W), jnp.float32),
        mesh=_mesh,
        scratch_types=[
            pltpu.VMEM((KB, CH), jnp.int32),
            pltpu.VMEM((KB, CH), jnp.int32),
            pltpu.VMEM((KB, CH), jnp.int32),
            pltpu.VMEM((KB, CH), jnp.int32),
            pltpu.VMEM((CH, W), jnp.float32),
            pltpu.VMEM((CH, W), jnp.float32),
            pltpu.VMEM_SHARED((NP, W), jnp.float32),
            pltpu.VMEM_SHARED((NP, W), jnp.float32),
            pltpu.SemaphoreType.DMA,
            pltpu.SemaphoreType.DMA,
            pltpu.SemaphoreType.DMA,
            pltpu.SemaphoreType.DMA,
            pltpu.SemaphoreType.DMA,
            pltpu.SemaphoreType.DMA,
        ],
        compiler_params=_sc_params,
    )


_agg64 = _make_agg(64)
_agg32 = _make_agg(32)


R = 2048
GRID = NP // R
R3 = 2000
GRID3 = N // R3


def _dis(d_ref):
    deg = d_ref[0, :, 0:1] + d_ref[1, :, 0:1] + 1.0
    return lax.rsqrt(deg)


def _l1_body(x_ref, w_ref, d_ref, o_ref):
    h = jnp.dot(x_ref[...], w_ref[...], preferred_element_type=jnp.float32)
    o_ref[...] = h * _dis(d_ref)


def _l1(xp, W1, degp):
    return pl.pallas_call(
        _l1_body,
        grid=(GRID,),
        in_specs=[
            pl.BlockSpec((R, 128), lambda i: (i, 0)),
            pl.BlockSpec((128, 64), lambda i: (0, 0)),
            pl.BlockSpec((2, R, 16), lambda i: (0, i, 0)),
        ],
        out_specs=pl.BlockSpec((R, 64), lambda i: (i, 0)),
        out_shape=jax.ShapeDtypeStruct((NP, 64), jnp.float32),
    )(xp, W1, degp)


def _l2_body(a_ref, g_ref, d_ref, b_ref, w_ref, o_ref):
    dis = _dis(d_ref)
    agg = a_ref[0] + a_ref[1] + g_ref[...]
    h = jnp.maximum(agg * dis + b_ref[...], 0.0)
    o_ref[...] = jnp.dot(h, w_ref[...], preferred_element_type=jnp.float32) * dis


def _l2(a1, g1, degp, b1, W2):
    return pl.pallas_call(
        _l2_body,
        grid=(GRID,),
        in_specs=[
            pl.BlockSpec((2, R, 64), lambda i: (0, i, 0)),
            pl.BlockSpec((R, 64), lambda i: (i, 0)),
            pl.BlockSpec((2, R, 16), lambda i: (0, i, 0)),
            pl.BlockSpec((1, 64), lambda i: (0, 0)),
            pl.BlockSpec((64, 32), lambda i: (0, 0)),
        ],
        out_specs=pl.BlockSpec((R, 32), lambda i: (i, 0)),
        out_shape=jax.ShapeDtypeStruct((NP, 32), jnp.float32),
    )(a1, g1, degp, b1, W2)


def _l3_body(a_ref, g_ref, d_ref, b_ref, o_ref):
    dis = _dis(d_ref)
    agg = a_ref[0] + a_ref[1] + g_ref[...]
    o_ref[...] = jnp.maximum(agg * dis + b_ref[...], 0.0)


def _l3(a2, g2, degp, b2):
    return pl.pallas_call(
        _l3_body,
        grid=(GRID3,),
        in_specs=[
            pl.BlockSpec((2, R3, 32), lambda i: (0, i, 0)),
            pl.BlockSpec((R3, 32), lambda i: (i, 0)),
            pl.BlockSpec((2, R3, 16), lambda i: (0, i, 0)),
            pl.BlockSpec((1, 32), lambda i: (0, 0)),
        ],
        out_specs=pl.BlockSpec((R3, 32), lambda i: (i, 0)),
        out_shape=jax.ShapeDtypeStruct((N, 32), jnp.float32),
    )(a2, g2, degp, b2)


def kernel(x, edge_index, W1, b1, W2, b2):
    ei = edge_index.astype(jnp.int32)
    eir = jnp.pad(ei, ((0, 0), (0, EP - E)),
                  constant_values=N).reshape(2, EP // CH, CH)
    xp = jnp.zeros((NP, 128), jnp.float32).at[:N].set(x)
    ones16 = jnp.ones((CH, 16), jnp.float32)
    z16 = jnp.zeros((NP, 16), jnp.float32)
    z64 = jnp.zeros((NP, 64), jnp.float32)
    z32 = jnp.zeros((NP, 32), jnp.float32)

    degp = _deg_call(eir, ones16, z16)
    g1 = _l1(xp, W1, degp)
    a1 = _agg64(g1, eir, z64)
    g2 = _l2(a1, g1, degp, b1.reshape(1, 64), W2)
    a2 = _agg32(g2, eir, z32)
    return _l3(a2, g2, degp, b2.reshape(1, 32))
